# Initial kernel scaffold; baseline (speedup 1.0000x reference)
import jax
import jax.numpy as jnp
from jax.experimental import pallas as pl


def kernel(criteria, edge_index, unlabeled_mask):
    raise NotImplementedError("write your pallas kernel here")


# speedup vs baseline: 218.6122x; 218.6122x over previous
import functools

import jax
import jax.numpy as jnp
from jax import lax
from jax.experimental import pallas as pl
from jax.experimental.pallas import tpu as pltpu
from jax.experimental.pallas import tpu_sc as plsc

N_NODES = 100000
N_EDGES = 6400000
BATCH = 64
BETA = 0.4

NPAD = 100352
ROWS = NPAD // 128
N_TILES = 32
EPT = N_EDGES // N_TILES
CHUNK = 2000
NCHUNK = EPT // CHUNK
W_WORDS = 3200


@functools.lru_cache(maxsize=None)
def _sc_segment_histogram():
    mesh = plsc.VectorSubcoreMesh(core_axis_name="c", subcore_axis_name="s")

    @functools.partial(
        pl.kernel,
        mesh=mesh,
        compiler_params=pltpu.CompilerParams(needs_layout_passes=False),
        out_type=jax.ShapeDtypeStruct((N_TILES * NPAD,), jnp.float32),
        scratch_types=[
            pltpu.VMEM((W_WORDS,), jnp.int32),
            pltpu.VMEM((NPAD,), jnp.float32),
            pltpu.VMEM((CHUNK,), jnp.int32),
            pltpu.VMEM((CHUNK,), jnp.int32),
            pltpu.VMEM((CHUNK,), jnp.int32),
            pltpu.VMEM((CHUNK,), jnp.int32),
            pltpu.SemaphoreType.DMA,
            pltpu.SemaphoreType.DMA,
            pltpu.SemaphoreType.DMA,
            pltpu.SemaphoreType.DMA,
        ],
    )
    def k(e0_hbm, e1_hbm, words_hbm, out_hbm, words_v, agg_v,
          e0b0, e0b1, e1b0, e1b1, s00, s01, s10, s11):
        wid = lax.axis_index("s") * 2 + lax.axis_index("c")
        base = wid * EPT

        pltpu.sync_copy(words_hbm, words_v)

        def zero_body(i, _):
            agg_v[pl.ds(i * 16, 16)] = jnp.zeros((16,), jnp.float32)
            return 0
        lax.fori_loop(0, NPAD // 16, zero_body, 0)

        e0bufs = (e0b0, e0b1)
        e1bufs = (e1b0, e1b1)
        sems0 = (s00, s01)
        sems1 = (s10, s11)

        def start(kk, b):
            off = base + kk * CHUNK
            pltpu.async_copy(e0_hbm.at[pl.ds(off, CHUNK)], e0bufs[b], sems0[b])
            pltpu.async_copy(e1_hbm.at[pl.ds(off, CHUNK)], e1bufs[b], sems1[b])

        def wait(kk, b):
            off = base + kk * CHUNK
            pltpu.make_async_copy(e0_hbm.at[pl.ds(off, CHUNK)], e0bufs[b],
                                  sems0[b]).wait()
            pltpu.make_async_copy(e1_hbm.at[pl.ds(off, CHUNK)], e1bufs[b],
                                  sems1[b]).wait()

        def process(b):
            e0b, e1b = e0bufs[b], e1bufs[b]
            def body(j, _):
                e1v = e1b[pl.ds(j * 16, 16)]
                wv = plsc.load_gather(words_v, [e1v >> 5])
                bit = (wv >> (e1v & 31)) & 1
                e0v = e0b[pl.ds(j * 16, 16)]
                plsc.addupdate_scatter(agg_v, [e0v], bit.astype(jnp.float32))
                return 0
            lax.fori_loop(0, CHUNK // 16, body, 0)

        start(0, 0)

        def outer(k2, _):
            kk = k2 * 2
            @pl.when(kk + 1 < NCHUNK)
            def _():
                start(kk + 1, 1)
            wait(kk, 0)
            process(0)
            @pl.when(kk + 1 < NCHUNK)
            def _():
                @pl.when(kk + 2 < NCHUNK)
                def _():
                    start(kk + 2, 0)
                wait(kk + 1, 1)
                process(1)
            return 0
        lax.fori_loop(0, (NCHUNK + 1) // 2, outer, 0)

        pltpu.sync_copy(agg_v, out_hbm.at[pl.ds(wid * NPAD, NPAD)])

    return k


def _tc_epilogue(crit_ref, unl_ref, parts_ref, vals_ref, ids_ref):
    idx2d = (lax.broadcasted_iota(jnp.int32, (ROWS, 128), 0) * 128
             + lax.broadcasted_iota(jnp.int32, (ROWS, 128), 1))
    valid = idx2d < N_NODES

    agg = jnp.sum(parts_ref[...], axis=0)

    craw = crit_ref[...]
    big = jnp.float32(3.0e38)
    cmin = jnp.min(jnp.where(valid, craw, big))
    cmax = jnp.max(jnp.where(valid, craw, -big))
    c = (craw - cmin) / (cmax - cmin)

    lraw = jnp.log(agg + 1e-5)
    lmin = jnp.min(jnp.where(valid, lraw, big))
    lmax = jnp.max(jnp.where(valid, lraw, -big))
    conf = (lraw - lmin) / (lmax - lmin)

    crit = ((1.0 - BETA) * c + BETA * conf) * unl_ref[...]
    crit = jnp.where(valid, crit, -big)

    col64 = lax.broadcasted_iota(jnp.int32, (1, 64), 1)

    def body(t, carry):
        x, vals, ids = carry
        m = jnp.max(x)
        cand = jnp.min(jnp.where(x == m, idx2d, jnp.int32(2**31 - 1)))
        vals = jnp.where(col64 == t, m, vals)
        ids = jnp.where(col64 == t, cand, ids)
        x = jnp.where(idx2d == cand, -big, x)
        return x, vals, ids

    _, vals, ids = lax.fori_loop(
        0, BATCH, body,
        (crit, jnp.zeros((1, 64), jnp.float32), jnp.zeros((1, 64), jnp.int32)))
    vals_ref[...] = vals
    ids_ref[...] = ids


def kernel(criteria, edge_index, unlabeled_mask):
    e0 = edge_index[0].astype(jnp.int32)
    e1 = edge_index[1].astype(jnp.int32)

    bits = unlabeled_mask.astype(jnp.int32)
    words = jnp.sum(bits.reshape(3125, 32) << jnp.arange(32, dtype=jnp.int32)[None, :],
                    axis=1).astype(jnp.int32)
    words = jnp.concatenate([words, jnp.zeros((W_WORDS - 3125,), jnp.int32)])

    parts = _sc_segment_histogram()(e0, e1, words).reshape(N_TILES, ROWS, 128)

    pad = NPAD - N_NODES
    crit2d = jnp.concatenate([criteria, jnp.zeros((pad,), jnp.float32)]).reshape(ROWS, 128)
    unl2d = jnp.concatenate([unlabeled_mask.astype(jnp.float32),
                             jnp.zeros((pad,), jnp.float32)]).reshape(ROWS, 128)

    vals, ids = pl.pallas_call(
        _tc_epilogue,
        out_shape=[
            jax.ShapeDtypeStruct((1, 64), jnp.float32),
            jax.ShapeDtypeStruct((1, 64), jnp.int32),
        ],
    )(crit2d, unl2d, parts)
    return vals.reshape(64), ids.reshape(64)


# speedup vs baseline: 252.5560x; 1.1553x over previous
import functools

import jax
import jax.numpy as jnp
from jax import lax
from jax.experimental import pallas as pl
from jax.experimental.pallas import tpu as pltpu
from jax.experimental.pallas import tpu_sc as plsc

N_NODES = 100000
N_EDGES = 6400000
BATCH = 64
BETA = 0.4

NPAD = 100352
ROWS = NPAD // 128
N_TILES = 32
EPT = N_EDGES // N_TILES
CHUNK = 4000
NCHUNK = EPT // CHUNK
UNROLL = 10
W_WORDS = 3200


@functools.lru_cache(maxsize=None)
def _sc_segment_histogram():
    mesh = plsc.VectorSubcoreMesh(core_axis_name="c", subcore_axis_name="s")

    @functools.partial(
        pl.kernel,
        mesh=mesh,
        compiler_params=pltpu.CompilerParams(needs_layout_passes=False),
        out_type=jax.ShapeDtypeStruct((N_TILES * NPAD,), jnp.int32),
        scratch_types=[
            pltpu.VMEM((W_WORDS,), jnp.int32),
            pltpu.VMEM((NPAD,), jnp.int32),
            pltpu.VMEM((CHUNK,), jnp.int32),
            pltpu.VMEM((CHUNK,), jnp.int32),
            pltpu.VMEM((CHUNK,), jnp.int32),
            pltpu.VMEM((CHUNK,), jnp.int32),
            pltpu.SemaphoreType.DMA,
            pltpu.SemaphoreType.DMA,
            pltpu.SemaphoreType.DMA,
            pltpu.SemaphoreType.DMA,
        ],
    )
    def k(e0_hbm, e1_hbm, words_hbm, out_hbm, words_v, agg_v,
          e0b0, e0b1, e1b0, e1b1, s00, s01, s10, s11):
        wid = lax.axis_index("s") * 2 + lax.axis_index("c")
        base = wid * EPT

        pltpu.sync_copy(words_hbm, words_v)

        zvec = jnp.zeros((16,), jnp.int32)

        def zero_body(i, _):
            for u in range(16):
                agg_v[pl.ds(i * 256 + u * 16, 16)] = zvec
            return 0
        lax.fori_loop(0, NPAD // 256, zero_body, 0)

        e0bufs = (e0b0, e0b1)
        e1bufs = (e1b0, e1b1)
        sems0 = (s00, s01)
        sems1 = (s10, s11)

        def start(kk, b):
            off = base + kk * CHUNK
            pltpu.async_copy(e0_hbm.at[pl.ds(off, CHUNK)], e0bufs[b], sems0[b])
            pltpu.async_copy(e1_hbm.at[pl.ds(off, CHUNK)], e1bufs[b], sems1[b])

        def wait(kk, b):
            off = base + kk * CHUNK
            pltpu.make_async_copy(e0_hbm.at[pl.ds(off, CHUNK)], e0bufs[b],
                                  sems0[b]).wait()
            pltpu.make_async_copy(e1_hbm.at[pl.ds(off, CHUNK)], e1bufs[b],
                                  sems1[b]).wait()

        def process(b):
            e0b, e1b = e0bufs[b], e1bufs[b]
            def body(j, _):
                base_j = j * (UNROLL * 16)
                for u in range(UNROLL):
                    off = base_j + u * 16
                    e1v = e1b[pl.ds(off, 16)]
                    wv = plsc.load_gather(words_v, [e1v >> 5])
                    bit = (wv >> (e1v & 31)) & 1
                    e0v = e0b[pl.ds(off, 16)]
                    plsc.addupdate_scatter(agg_v, [e0v], bit)
                return 0
            lax.fori_loop(0, CHUNK // (UNROLL * 16), body, 0)

        start(0, 0)

        def outer(k2, _):
            kk = k2 * 2
            @pl.when(kk + 1 < NCHUNK)
            def _():
                start(kk + 1, 1)
            wait(kk, 0)
            process(0)
            @pl.when(kk + 1 < NCHUNK)
            def _():
                @pl.when(kk + 2 < NCHUNK)
                def _():
                    start(kk + 2, 0)
                wait(kk + 1, 1)
                process(1)
            return 0
        lax.fori_loop(0, (NCHUNK + 1) // 2, outer, 0)

        pltpu.sync_copy(agg_v, out_hbm.at[pl.ds(wid * NPAD, NPAD)])

    return k


def _tc_epilogue(crit_ref, unl_ref, parts_ref, vals_ref, ids_ref):
    idx2d = (lax.broadcasted_iota(jnp.int32, (ROWS, 128), 0) * 128
             + lax.broadcasted_iota(jnp.int32, (ROWS, 128), 1))
    valid = idx2d < N_NODES

    agg = jnp.sum(parts_ref[...], axis=0).astype(jnp.float32)

    craw = crit_ref[...]
    big = jnp.float32(3.0e38)
    cmin = jnp.min(jnp.where(valid, craw, big))
    cmax = jnp.max(jnp.where(valid, craw, -big))
    c = (craw - cmin) / (cmax - cmin)

    lraw = jnp.log(agg + 1e-5)
    lmin = jnp.min(jnp.where(valid, lraw, big))
    lmax = jnp.max(jnp.where(valid, lraw, -big))
    conf = (lraw - lmin) / (lmax - lmin)

    crit = ((1.0 - BETA) * c + BETA * conf) * unl_ref[...]
    crit = jnp.where(valid, crit, -big)

    col64 = lax.broadcasted_iota(jnp.int32, (1, 64), 1)

    def body(t, carry):
        x, vals, ids = carry
        m = jnp.max(x)
        cand = jnp.min(jnp.where(x == m, idx2d, jnp.int32(2**31 - 1)))
        vals = jnp.where(col64 == t, m, vals)
        ids = jnp.where(col64 == t, cand, ids)
        x = jnp.where(idx2d == cand, -big, x)
        return x, vals, ids

    _, vals, ids = lax.fori_loop(
        0, BATCH, body,
        (crit, jnp.zeros((1, 64), jnp.float32), jnp.zeros((1, 64), jnp.int32)))
    vals_ref[...] = vals
    ids_ref[...] = ids


def kernel(criteria, edge_index, unlabeled_mask):
    e0 = edge_index[0].astype(jnp.int32)
    e1 = edge_index[1].astype(jnp.int32)

    bits = unlabeled_mask.astype(jnp.int32)
    words = jnp.sum(bits.reshape(3125, 32) << jnp.arange(32, dtype=jnp.int32)[None, :],
                    axis=1).astype(jnp.int32)
    words = jnp.concatenate([words, jnp.zeros((W_WORDS - 3125,), jnp.int32)])

    parts = _sc_segment_histogram()(e0, e1, words).reshape(N_TILES, ROWS, 128)

    pad = NPAD - N_NODES
    crit2d = jnp.concatenate([criteria, jnp.zeros((pad,), jnp.float32)]).reshape(ROWS, 128)
    unl2d = jnp.concatenate([unlabeled_mask.astype(jnp.float32),
                             jnp.zeros((pad,), jnp.float32)]).reshape(ROWS, 128)

    vals, ids = pl.pallas_call(
        _tc_epilogue,
        out_shape=[
            jax.ShapeDtypeStruct((1, 64), jnp.float32),
            jax.ShapeDtypeStruct((1, 64), jnp.int32),
        ],
    )(crit2d, unl2d, parts)
    return vals.reshape(64), ids.reshape(64)


# speedup vs baseline: 413.0385x; 1.6354x over previous
import functools

import jax
import jax.numpy as jnp
from jax import lax
from jax.experimental import pallas as pl
from jax.experimental.pallas import tpu as pltpu
from jax.experimental.pallas import tpu_sc as plsc

N_NODES = 100000
N_EDGES = 6400000
BATCH = 64
BETA = 0.4

NPAD = 100352
ROWS = NPAD // 128
N_TILES = 32
EPT = N_EDGES // N_TILES
CHUNK = 4000
NCHUNK = EPT // CHUNK
UNROLL = 10
W_WORDS = 3200


@functools.lru_cache(maxsize=None)
def _sc_segment_histogram():
    mesh = plsc.VectorSubcoreMesh(core_axis_name="c", subcore_axis_name="s")

    @functools.partial(
        pl.kernel,
        mesh=mesh,
        compiler_params=pltpu.CompilerParams(needs_layout_passes=False),
        out_type=jax.ShapeDtypeStruct((N_TILES * NPAD,), jnp.int32),
        scratch_types=[
            pltpu.VMEM((W_WORDS,), jnp.int32),
            pltpu.VMEM((NPAD,), jnp.int32),
            pltpu.VMEM((CHUNK,), jnp.int32),
            pltpu.VMEM((CHUNK,), jnp.int32),
            pltpu.VMEM((CHUNK,), jnp.int32),
            pltpu.VMEM((CHUNK,), jnp.int32),
            pltpu.SemaphoreType.DMA,
            pltpu.SemaphoreType.DMA,
            pltpu.SemaphoreType.DMA,
            pltpu.SemaphoreType.DMA,
        ],
    )
    def k(e0_hbm, e1_hbm, words_hbm, out_hbm, words_v, agg_v,
          e0b0, e0b1, e1b0, e1b1, s00, s01, s10, s11):
        wid = lax.axis_index("s") * 2 + lax.axis_index("c")
        base = wid * EPT

        pltpu.sync_copy(words_hbm, words_v)

        zvec = jnp.zeros((16,), jnp.int32)

        def zero_body(i, _):
            for u in range(16):
                agg_v[pl.ds(i * 256 + u * 16, 16)] = zvec
            return 0
        lax.fori_loop(0, NPAD // 256, zero_body, 0)

        e0bufs = (e0b0, e0b1)
        e1bufs = (e1b0, e1b1)
        sems0 = (s00, s01)
        sems1 = (s10, s11)

        def start(kk, b):
            off = base + kk * CHUNK
            pltpu.async_copy(e0_hbm.at[pl.ds(off, CHUNK)], e0bufs[b], sems0[b])
            pltpu.async_copy(e1_hbm.at[pl.ds(off, CHUNK)], e1bufs[b], sems1[b])

        def wait(kk, b):
            off = base + kk * CHUNK
            pltpu.make_async_copy(e0_hbm.at[pl.ds(off, CHUNK)], e0bufs[b],
                                  sems0[b]).wait()
            pltpu.make_async_copy(e1_hbm.at[pl.ds(off, CHUNK)], e1bufs[b],
                                  sems1[b]).wait()

        def process(b):
            e0b, e1b = e0bufs[b], e1bufs[b]
            def body(j, _):
                base_j = j * (UNROLL * 16)
                offs = [base_j + u * 16 for u in range(UNROLL)]
                e1vs = [e1b[pl.ds(o, 16)] for o in offs]
                e0vs = [e0b[pl.ds(o, 16)] for o in offs]
                wvs = [plsc.load_gather(words_v, [v >> 5]) for v in e1vs]
                bits = [(w >> (v & 31)) & 1 for w, v in zip(wvs, e1vs)]
                for e0v, bit in zip(e0vs, bits):
                    plsc.addupdate_scatter(agg_v, [e0v], bit)
                return 0
            lax.fori_loop(0, CHUNK // (UNROLL * 16), body, 0)

        start(0, 0)

        def outer(k2, _):
            kk = k2 * 2
            @pl.when(kk + 1 < NCHUNK)
            def _():
                start(kk + 1, 1)
            wait(kk, 0)
            process(0)
            @pl.when(kk + 1 < NCHUNK)
            def _():
                @pl.when(kk + 2 < NCHUNK)
                def _():
                    start(kk + 2, 0)
                wait(kk + 1, 1)
                process(1)
            return 0
        lax.fori_loop(0, (NCHUNK + 1) // 2, outer, 0)

        pltpu.sync_copy(agg_v, out_hbm.at[pl.ds(wid * NPAD, NPAD)])

    return k


def _tc_epilogue(crit_ref, unl_ref, parts_ref, vals_ref, ids_ref):
    idx2d = (lax.broadcasted_iota(jnp.int32, (ROWS, 128), 0) * 128
             + lax.broadcasted_iota(jnp.int32, (ROWS, 128), 1))
    valid = idx2d < N_NODES

    agg = jnp.sum(parts_ref[...], axis=0).astype(jnp.float32)

    craw = crit_ref[...]
    big = jnp.float32(3.0e38)
    cmin = jnp.min(jnp.where(valid, craw, big))
    cmax = jnp.max(jnp.where(valid, craw, -big))
    c = (craw - cmin) / (cmax - cmin)

    lraw = jnp.log(agg + 1e-5)
    lmin = jnp.min(jnp.where(valid, lraw, big))
    lmax = jnp.max(jnp.where(valid, lraw, -big))
    conf = (lraw - lmin) / (lmax - lmin)

    crit = ((1.0 - BETA) * c + BETA * conf) * unl_ref[...]
    crit = jnp.where(valid, crit, -big)

    col64 = lax.broadcasted_iota(jnp.int32, (1, 64), 1)

    def body(t, carry):
        x, vals, ids = carry
        m = jnp.max(x)
        cand = jnp.min(jnp.where(x == m, idx2d, jnp.int32(2**31 - 1)))
        vals = jnp.where(col64 == t, m, vals)
        ids = jnp.where(col64 == t, cand, ids)
        x = jnp.where(idx2d == cand, -big, x)
        return x, vals, ids

    _, vals, ids = lax.fori_loop(
        0, BATCH, body,
        (crit, jnp.zeros((1, 64), jnp.float32), jnp.zeros((1, 64), jnp.int32)))
    vals_ref[...] = vals
    ids_ref[...] = ids


def kernel(criteria, edge_index, unlabeled_mask):
    e0 = edge_index[0].astype(jnp.int32)
    e1 = edge_index[1].astype(jnp.int32)

    bits = unlabeled_mask.astype(jnp.int32)
    words = jnp.sum(bits.reshape(3125, 32) << jnp.arange(32, dtype=jnp.int32)[None, :],
                    axis=1).astype(jnp.int32)
    words = jnp.concatenate([words, jnp.zeros((W_WORDS - 3125,), jnp.int32)])

    parts = _sc_segment_histogram()(e0, e1, words).reshape(N_TILES, ROWS, 128)

    pad = NPAD - N_NODES
    crit2d = jnp.concatenate([criteria, jnp.zeros((pad,), jnp.float32)]).reshape(ROWS, 128)
    unl2d = jnp.concatenate([unlabeled_mask.astype(jnp.float32),
                             jnp.zeros((pad,), jnp.float32)]).reshape(ROWS, 128)

    vals, ids = pl.pallas_call(
        _tc_epilogue,
        out_shape=[
            jax.ShapeDtypeStruct((1, 64), jnp.float32),
            jax.ShapeDtypeStruct((1, 64), jnp.int32),
        ],
    )(crit2d, unl2d, parts)
    return vals.reshape(64), ids.reshape(64)


# speedup vs baseline: 463.1260x; 1.1213x over previous
import functools

import jax
import jax.numpy as jnp
from jax import lax
from jax.experimental import pallas as pl
from jax.experimental.pallas import tpu as pltpu
from jax.experimental.pallas import tpu_sc as plsc

N_NODES = 100000
N_EDGES = 6400000
BATCH = 64
BETA = 0.4

NPAD = 100352
ROWS = NPAD // 128
C1R = ROWS // 8
N_TILES = 32
EPT = N_EDGES // N_TILES
CHUNK = 4000
NCHUNK = EPT // CHUNK
UNROLL = 10
W_WORDS = 3200


@functools.lru_cache(maxsize=None)
def _sc_segment_histogram():
    mesh = plsc.VectorSubcoreMesh(core_axis_name="c", subcore_axis_name="s")

    @functools.partial(
        pl.kernel,
        mesh=mesh,
        compiler_params=pltpu.CompilerParams(needs_layout_passes=False),
        out_type=jax.ShapeDtypeStruct((N_TILES * NPAD,), jnp.int32),
        scratch_types=[
            pltpu.VMEM((W_WORDS,), jnp.int32),
            pltpu.VMEM((NPAD,), jnp.int32),
            pltpu.VMEM((CHUNK,), jnp.int32),
            pltpu.VMEM((CHUNK,), jnp.int32),
            pltpu.VMEM((CHUNK,), jnp.int32),
            pltpu.VMEM((CHUNK,), jnp.int32),
            pltpu.SemaphoreType.DMA,
            pltpu.SemaphoreType.DMA,
            pltpu.SemaphoreType.DMA,
            pltpu.SemaphoreType.DMA,
        ],
    )
    def k(edges_hbm, words_hbm, out_hbm, words_v, agg_v,
          e0b0, e0b1, e1b0, e1b1, s00, s01, s10, s11):
        wid = lax.axis_index("s") * 2 + lax.axis_index("c")
        base = wid * EPT

        pltpu.sync_copy(words_hbm, words_v)

        zvec = jnp.zeros((16,), jnp.int32)

        def zero_body(i, _):
            for u in range(16):
                agg_v[pl.ds(i * 256 + u * 16, 16)] = zvec
            return 0
        lax.fori_loop(0, NPAD // 256, zero_body, 0)

        e0bufs = (e0b0, e0b1)
        e1bufs = (e1b0, e1b1)
        sems0 = (s00, s01)
        sems1 = (s10, s11)

        def start(kk, b):
            off = base + kk * CHUNK
            pltpu.async_copy(edges_hbm.at[pl.ds(off, CHUNK)],
                             e0bufs[b], sems0[b])
            pltpu.async_copy(edges_hbm.at[pl.ds(N_EDGES + off, CHUNK)],
                             e1bufs[b], sems1[b])

        def wait(kk, b):
            off = base + kk * CHUNK
            pltpu.make_async_copy(edges_hbm.at[pl.ds(off, CHUNK)],
                                  e0bufs[b], sems0[b]).wait()
            pltpu.make_async_copy(edges_hbm.at[pl.ds(N_EDGES + off, CHUNK)],
                                  e1bufs[b], sems1[b]).wait()

        def process(b):
            e0b, e1b = e0bufs[b], e1bufs[b]
            def body(j, _):
                base_j = j * (UNROLL * 16)
                offs = [base_j + u * 16 for u in range(UNROLL)]
                e1vs = [e1b[pl.ds(o, 16)] for o in offs]
                e0vs = [e0b[pl.ds(o, 16)] for o in offs]
                wvs = [plsc.load_gather(words_v, [v >> 5]) for v in e1vs]
                bits = [(w >> (v & 31)) & 1 for w, v in zip(wvs, e1vs)]
                for e0v, bit in zip(e0vs, bits):
                    plsc.addupdate_scatter(agg_v, [e0v], bit)
                return 0
            lax.fori_loop(0, CHUNK // (UNROLL * 16), body, 0)

        start(0, 0)

        def outer(k2, _):
            kk = k2 * 2
            @pl.when(kk + 1 < NCHUNK)
            def _():
                start(kk + 1, 1)
            wait(kk, 0)
            process(0)
            @pl.when(kk + 1 < NCHUNK)
            def _():
                @pl.when(kk + 2 < NCHUNK)
                def _():
                    start(kk + 2, 0)
                wait(kk + 1, 1)
                process(1)
            return 0
        lax.fori_loop(0, (NCHUNK + 1) // 2, outer, 0)

        pltpu.sync_copy(agg_v, out_hbm.at[pl.ds(wid * NPAD, NPAD)])

    return k


def _tc_epilogue(crit_ref, unl_ref, parts_ref, vals_ref, ids_ref,
                 x_ref, c1_ref):
    idx2d = (lax.broadcasted_iota(jnp.int32, (ROWS, 128), 0) * 128
             + lax.broadcasted_iota(jnp.int32, (ROWS, 128), 1))
    valid = idx2d < N_NODES

    agg = jnp.sum(parts_ref[...], axis=0).astype(jnp.float32)

    craw = crit_ref[...]
    big = jnp.float32(3.0e38)
    cmin = jnp.min(jnp.where(valid, craw, big))
    cmax = jnp.max(jnp.where(valid, craw, -big))
    c = (craw - cmin) / (cmax - cmin)

    lraw = jnp.log(agg + 1e-5)
    lmin = jnp.min(jnp.where(valid, lraw, big))
    lmax = jnp.max(jnp.where(valid, lraw, -big))
    conf = (lraw - lmin) / (lmax - lmin)

    crit = ((1.0 - BETA) * c + BETA * conf) * unl_ref[...]
    crit = jnp.where(valid, crit, -big)
    x_ref[...] = crit

    def c1_body(r, _):
        c1_ref[pl.ds(r, 1), :] = jnp.max(
            x_ref[pl.ds(r * 8, 8), :], axis=0, keepdims=True)
        return 0
    lax.fori_loop(0, C1R, c1_body, 0)

    col64 = lax.broadcasted_iota(jnp.int32, (1, 64), 1)
    row98 = lax.broadcasted_iota(jnp.int32, (C1R, 128), 0)
    bidx_loc = (lax.broadcasted_iota(jnp.int32, (8, 128), 0) * 128
                + lax.broadcasted_iota(jnp.int32, (8, 128), 1))
    imax = jnp.int32(2**31 - 1)

    def body(t, carry):
        vals, ids = carry
        c1 = c1_ref[...]
        m = jnp.max(c1)
        b = jnp.min(jnp.where(c1 == m, row98, jnp.int32(C1R)))
        blk = x_ref[pl.ds(b * 8, 8), :]
        bidx = b * 1024 + bidx_loc
        cand = jnp.min(jnp.where(blk == m, bidx, imax))
        vals = jnp.where(col64 == t, m, vals)
        ids = jnp.where(col64 == t, cand, ids)
        nblk = jnp.where(bidx == cand, -big, blk)
        x_ref[pl.ds(b * 8, 8), :] = nblk
        c1_ref[pl.ds(b, 1), :] = jnp.max(nblk, axis=0, keepdims=True)
        return vals, ids

    vals, ids = lax.fori_loop(
        0, BATCH, body,
        (jnp.zeros((1, 64), jnp.float32), jnp.zeros((1, 64), jnp.int32)))
    vals_ref[...] = vals
    ids_ref[...] = ids


def kernel(criteria, edge_index, unlabeled_mask):
    edges = edge_index.astype(jnp.int32).reshape(2 * N_EDGES)

    bits = unlabeled_mask.astype(jnp.int32)
    words = jnp.sum(bits.reshape(3125, 32) << jnp.arange(32, dtype=jnp.int32)[None, :],
                    axis=1).astype(jnp.int32)
    words = jnp.concatenate([words, jnp.zeros((W_WORDS - 3125,), jnp.int32)])

    parts = _sc_segment_histogram()(edges, words).reshape(N_TILES, ROWS, 128)

    pad = NPAD - N_NODES
    crit2d = jnp.concatenate([criteria, jnp.zeros((pad,), jnp.float32)]).reshape(ROWS, 128)
    unl2d = jnp.concatenate([unlabeled_mask.astype(jnp.float32),
                             jnp.zeros((pad,), jnp.float32)]).reshape(ROWS, 128)

    vals, ids = pl.pallas_call(
        _tc_epilogue,
        out_shape=[
            jax.ShapeDtypeStruct((1, 64), jnp.float32),
            jax.ShapeDtypeStruct((1, 64), jnp.int32),
        ],
        scratch_shapes=[
            pltpu.VMEM((ROWS, 128), jnp.float32),
            pltpu.VMEM((C1R, 128), jnp.float32),
        ],
    )(crit2d, unl2d, parts)
    return vals.reshape(64), ids.reshape(64)


# speedup vs baseline: 541.0339x; 1.1682x over previous
import functools

import jax
import jax.numpy as jnp
from jax import lax
from jax.experimental import pallas as pl
from jax.experimental.pallas import tpu as pltpu
from jax.experimental.pallas import tpu_sc as plsc

N_NODES = 100000
N_EDGES = 6400000
BATCH = 64
BETA = 0.4

NPAD = 100352
ROWS = NPAD // 128
C1R = ROWS // 8
N_TILES = 32
CHUNK = 2048
CHUNK_W = 2 * CHUNK
NCHUNKS = N_EDGES // CHUNK
UNROLL = 8
W_WORDS = 3200


@functools.lru_cache(maxsize=None)
def _sc_segment_histogram():
    mesh = plsc.VectorSubcoreMesh(core_axis_name="c", subcore_axis_name="s")

    @functools.partial(
        pl.kernel,
        mesh=mesh,
        compiler_params=pltpu.CompilerParams(needs_layout_passes=False),
        out_type=jax.ShapeDtypeStruct((N_TILES * NPAD,), jnp.int32),
        scratch_types=[
            pltpu.VMEM((W_WORDS,), jnp.int32),
            pltpu.VMEM((NPAD,), jnp.int32),
            pltpu.VMEM((CHUNK_W,), jnp.int32),
            pltpu.VMEM((CHUNK_W,), jnp.int32),
            pltpu.SemaphoreType.DMA,
            pltpu.SemaphoreType.DMA,
        ],
    )
    def k(edges_hbm, words_hbm, out_hbm, words_v, agg_v,
          eb0, eb1, s0, s1):
        wid = lax.axis_index("s") * 2 + lax.axis_index("c")
        nck = 97 + jnp.where(wid < NCHUNKS - 32 * 97, 1, 0)

        pltpu.sync_copy(words_hbm, words_v)

        zvec = jnp.zeros((16,), jnp.int32)

        def zero_body(i, _):
            for u in range(16):
                agg_v[pl.ds(i * 256 + u * 16, 16)] = zvec
            return 0
        lax.fori_loop(0, NPAD // 256, zero_body, 0)

        ebufs = (eb0, eb1)
        sems = (s0, s1)

        def start(kk, b):
            off = (wid + kk * N_TILES) * CHUNK_W
            pltpu.async_copy(edges_hbm.at[pl.ds(off, CHUNK_W)],
                             ebufs[b], sems[b])

        def wait(kk, b):
            off = (wid + kk * N_TILES) * CHUNK_W
            pltpu.make_async_copy(edges_hbm.at[pl.ds(off, CHUNK_W)],
                                  ebufs[b], sems[b]).wait()

        def process(b):
            eb = ebufs[b]
            def body(j, _):
                offs = [j * 256 + u * 16 for u in range(UNROLL)]
                e1vs = [eb[pl.ds(o + 128, 16)] for o in offs]
                e0vs = [eb[pl.ds(o, 16)] for o in offs]
                wvs = [plsc.load_gather(words_v, [v >> 5]) for v in e1vs]
                bits = [(w >> (v & 31)) & 1 for w, v in zip(wvs, e1vs)]
                for e0v, bit in zip(e0vs, bits):
                    plsc.addupdate_scatter(agg_v, [e0v], bit)
                return 0
            lax.fori_loop(0, CHUNK // (UNROLL * 16), body, 0)

        start(0, 0)

        def outer(k2, _):
            kk = k2 * 2
            @pl.when(kk + 1 < nck)
            def _():
                start(kk + 1, 1)
            wait(kk, 0)
            process(0)
            @pl.when(kk + 1 < nck)
            def _():
                @pl.when(kk + 2 < nck)
                def _():
                    start(kk + 2, 0)
                wait(kk + 1, 1)
                process(1)
            return 0
        lax.fori_loop(0, 49, outer, 0)

        pltpu.sync_copy(agg_v, out_hbm.at[pl.ds(wid * NPAD, NPAD)])

    return k


def _tc_epilogue(crit_ref, unl_ref, parts_ref, vals_ref, ids_ref,
                 x_ref, c1_ref):
    idx2d = (lax.broadcasted_iota(jnp.int32, (ROWS, 128), 0) * 128
             + lax.broadcasted_iota(jnp.int32, (ROWS, 128), 1))
    valid = idx2d < N_NODES

    agg = jnp.sum(parts_ref[...], axis=0).astype(jnp.float32)

    craw = crit_ref[...]
    big = jnp.float32(3.0e38)
    cmin = jnp.min(jnp.where(valid, craw, big))
    cmax = jnp.max(jnp.where(valid, craw, -big))
    c = (craw - cmin) / (cmax - cmin)

    lraw = jnp.log(agg + 1e-5)
    lmin = jnp.min(jnp.where(valid, lraw, big))
    lmax = jnp.max(jnp.where(valid, lraw, -big))
    conf = (lraw - lmin) / (lmax - lmin)

    crit = ((1.0 - BETA) * c + BETA * conf) * unl_ref[...]
    crit = jnp.where(valid, crit, -big)
    x_ref[...] = crit

    def c1_body(r, _):
        c1_ref[pl.ds(r, 1), :] = jnp.max(
            x_ref[pl.ds(r * 8, 8), :], axis=0, keepdims=True)
        return 0
    lax.fori_loop(0, C1R, c1_body, 0)

    col64 = lax.broadcasted_iota(jnp.int32, (1, 64), 1)
    row98 = lax.broadcasted_iota(jnp.int32, (C1R, 128), 0)
    bidx_loc = (lax.broadcasted_iota(jnp.int32, (8, 128), 0) * 128
                + lax.broadcasted_iota(jnp.int32, (8, 128), 1))
    imax = jnp.int32(2**31 - 1)

    def body(t, carry):
        vals, ids = carry
        c1 = c1_ref[...]
        m = jnp.max(c1)
        b = jnp.min(jnp.where(c1 == m, row98, jnp.int32(C1R)))
        blk = x_ref[pl.ds(b * 8, 8), :]
        bidx = b * 1024 + bidx_loc
        cand = jnp.min(jnp.where(blk == m, bidx, imax))
        vals = jnp.where(col64 == t, m, vals)
        ids = jnp.where(col64 == t, cand, ids)
        nblk = jnp.where(bidx == cand, -big, blk)
        x_ref[pl.ds(b * 8, 8), :] = nblk
        c1_ref[pl.ds(b, 1), :] = jnp.max(nblk, axis=0, keepdims=True)
        return vals, ids

    vals, ids = lax.fori_loop(
        0, BATCH, body,
        (jnp.zeros((1, 64), jnp.float32), jnp.zeros((1, 64), jnp.int32)))
    vals_ref[...] = vals
    ids_ref[...] = ids


def kernel(criteria, edge_index, unlabeled_mask):
    edges = jnp.transpose(
        edge_index.astype(jnp.int32).reshape(2, N_EDGES // 128, 128),
        (1, 0, 2)).reshape(2 * N_EDGES)

    bits = unlabeled_mask.astype(jnp.int32)
    words = jnp.sum(bits.reshape(3125, 32) << jnp.arange(32, dtype=jnp.int32)[None, :],
                    axis=1).astype(jnp.int32)
    words = jnp.concatenate([words, jnp.zeros((W_WORDS - 3125,), jnp.int32)])

    parts = _sc_segment_histogram()(edges, words).reshape(N_TILES, ROWS, 128)

    pad = NPAD - N_NODES
    crit2d = jnp.concatenate([criteria, jnp.zeros((pad,), jnp.float32)]).reshape(ROWS, 128)
    unl2d = jnp.concatenate([unlabeled_mask.astype(jnp.float32),
                             jnp.zeros((pad,), jnp.float32)]).reshape(ROWS, 128)

    vals, ids = pl.pallas_call(
        _tc_epilogue,
        out_shape=[
            jax.ShapeDtypeStruct((1, 64), jnp.float32),
            jax.ShapeDtypeStruct((1, 64), jnp.int32),
        ],
        scratch_shapes=[
            pltpu.VMEM((ROWS, 128), jnp.float32),
            pltpu.VMEM((C1R, 128), jnp.float32),
        ],
    )(crit2d, unl2d, parts)
    return vals.reshape(64), ids.reshape(64)
